# Initial kernel scaffold; baseline (speedup 1.0000x reference)
import jax
import jax.numpy as jnp
from jax.experimental import pallas as pl


def kernel(x_reaction, x_protein, edge_index_pr, edge_index_rp, emb_reaction, emb_protein, Wl_pr_0, bl_pr_0, Wr_pr_0, Wl_rp_0, bl_rp_0, Wr_rp_0, Wl_pr_1, bl_pr_1, Wr_pr_1, Wl_rp_1, bl_rp_1, Wr_rp_1, W_out, b_out):
    raise NotImplementedError("write your pallas kernel here")


# speedup vs baseline: 4.5111x; 4.5111x over previous
import functools

import jax
import jax.numpy as jnp
from jax import lax
from jax.experimental import pallas as pl
from jax.experimental.pallas import tpu as pltpu
from jax.experimental.pallas import tpu_sc as plsc

N_R = 10000
N_P = 50000
E = 320000
D = 128
OUT = 2

NC = 2
NS = 16
NW = NC * NS

R_PAD = 10240
P_PAD = 50176
K = 128

ECH = E // K
ECH_Q, ECH_R = divmod(ECH, NW)
KH = 64
PCH = P_PAD // KH
PCH_Q, PCH_R = divmod(PCH, NW)

FW = P_PAD // NW
SCH = 2048
SCH_N, SCH_T = divmod(E, SCH)

_MESH = plsc.VectorSubcoreMesh(
    core_axis_name="c", subcore_axis_name="s", num_cores=NC, num_subcores=NS)


def _worker_id():
  return lax.axis_index("s") * NC + lax.axis_index("c")


def _sc_hp0(xp_pad, emb_aug):
  @functools.partial(
      pl.kernel,
      out_type=jax.ShapeDtypeStruct((P_PAD, D), jnp.float32),
      mesh=_MESH,
      compiler_params=pltpu.CompilerParams(needs_layout_passes=False),
      scratch_types=[
          pltpu.VMEM((KH,), jnp.int32),
          pltpu.VMEM((KH, D), jnp.float32),
          pltpu.SemaphoreType.DMA,
      ],
  )
  def body(xp_hbm, emb_hbm, hp0_hbm, sidx, rows, sem):
    gw = _worker_id()
    nhp = PCH_Q + jnp.where(gw < PCH_R, 1, 0)

    @pl.loop(0, nhp)
    def _hp(i):
      off = (gw + i * NW) * KH
      pltpu.sync_copy(xp_hbm.at[pl.ds(off, KH)], sidx)
      pltpu.async_copy(emb_hbm.at[sidx], rows, sem).wait()
      pltpu.sync_copy(rows, hp0_hbm.at[pl.ds(off, KH)])

  return body(xp_pad, emb_aug)


def _sc_edge_agg(table, src, dst):
  @functools.partial(
      pl.kernel,
      out_type=jax.ShapeDtypeStruct((NC * R_PAD, D), jnp.float32),
      mesh=_MESH,
      compiler_params=pltpu.CompilerParams(needs_layout_passes=False),
      scratch_types=[
          pltpu.VMEM_SHARED((R_PAD, D), jnp.float32),
          pltpu.VMEM((K,), jnp.int32),
          pltpu.VMEM((K,), jnp.int32),
          pltpu.VMEM((K, D), jnp.float32),
          pltpu.SemaphoreType.DMA,
      ],
  )
  def body(tab_hbm, src_hbm, dst_hbm, agg_hbm, agg_s, sidx, didx, rows, sem):
    c = lax.axis_index("c")
    s = lax.axis_index("s")
    gw = _worker_id()

    @pl.loop(0, K)
    def _z(r):
      for j in range(D // 16):
        rows[r, pl.ds(j * 16, 16)] = jnp.zeros((16,), jnp.float32)

    for t in range(5):
      pltpu.sync_copy(rows, agg_s.at[pl.ds(s * 640 + t * K, K)])
    plsc.subcore_barrier()

    nch = ECH_Q + jnp.where(gw < ECH_R, 1, 0)

    @pl.loop(0, nch)
    def _edge(i):
      off = (gw + i * NW) * K
      pltpu.sync_copy(src_hbm.at[pl.ds(off, K)], sidx)
      pltpu.sync_copy(dst_hbm.at[pl.ds(off, K)], didx)
      pltpu.async_copy(tab_hbm.at[sidx], rows, sem).wait()
      pltpu.sync_copy(rows, agg_s.at[didx], add=True)

    plsc.subcore_barrier()
    for t in range(5):
      pltpu.sync_copy(agg_s.at[pl.ds(s * 640 + t * K, K)], rows)
      pltpu.sync_copy(rows, agg_hbm.at[pl.ds(c * R_PAD + s * 640 + t * K, K)])

  return body(table, src, dst)


def _sc_flags(dst_rp):
  @functools.partial(
      pl.kernel,
      out_type=jax.ShapeDtypeStruct((P_PAD,), jnp.float32),
      mesh=_MESH,
      compiler_params=pltpu.CompilerParams(needs_layout_passes=False),
      scratch_types=[
          pltpu.VMEM((SCH,), jnp.int32),
          pltpu.VMEM((FW + 16,), jnp.float32),
      ],
  )
  def body(drp_hbm, flag_hbm, dbuf, hist):
    gw = _worker_id()

    @pl.loop(0, (FW + 16) // 16)
    def _zh(r):
      hist[pl.ds(r * 16, 16)] = jnp.zeros((16,), jnp.float32)

    base = gw * FW

    def _mark(nvec):
      def mark(j):
        v = dbuf[pl.ds(j * 16, 16)]
        m = (v >= base) & (v < base + FW)
        idx = jnp.where(m, v - base, FW)
        plsc.store_scatter(hist, [idx], jnp.full((16,), 1.0, jnp.float32))
      return pl.loop(0, nvec)(mark)

    @pl.loop(0, SCH_N)
    def _scan(i):
      pltpu.sync_copy(drp_hbm.at[pl.ds(i * SCH, SCH)], dbuf)
      _mark(SCH // 16)

    pltpu.sync_copy(drp_hbm.at[pl.ds(SCH_N * SCH, SCH_T)], dbuf.at[pl.ds(0, SCH_T)])
    _mark(SCH_T // 16)

    pltpu.sync_copy(hist.at[pl.ds(0, FW)], flag_hbm.at[pl.ds(base, FW)])

  return body(dst_rp)


_PREC = lax.Precision.HIGHEST


def _l2norm(t):
  return t / jnp.maximum(jnp.sqrt(jnp.sum(t * t, axis=-1, keepdims=True)), 1e-12)


def _tc_protein_body(hp0_ref, flag_ref, emb_ref, Wl_ref, bl_ref, Wr_ref, out_ref):
  flag = lax.broadcast_in_dim(flag_ref[...], (1024, D), (0,))
  u = jnp.dot(emb_ref[...], Wl_ref[...], precision=_PREC)
  t = (flag * u + bl_ref[...]
       + jnp.dot(hp0_ref[...], Wr_ref[...], precision=_PREC))
  out_ref[...] = jnp.maximum(_l2norm(t), 0.0)


def _tc_protein(hp0, flag, emb_reaction, Wl, bl, Wr):
  grid = (P_PAD // 1024,)
  return pl.pallas_call(
      _tc_protein_body,
      grid=grid,
      in_specs=[
          pl.BlockSpec((1024, D), lambda i: (i, 0)),
          pl.BlockSpec((1024,), lambda i: (i,)),
          pl.BlockSpec((1, D), lambda i: (0, 0)),
          pl.BlockSpec((D, D), lambda i: (0, 0)),
          pl.BlockSpec((D,), lambda i: (0,)),
          pl.BlockSpec((D, D), lambda i: (0, 0)),
      ],
      out_specs=pl.BlockSpec((1024, D), lambda i: (i, 0)),
      out_shape=jax.ShapeDtypeStruct((P_PAD, D), jnp.float32),
  )(hp0, flag, emb_reaction, Wl, bl, Wr)


def _tc_react_body(agg_ref, cnt_ref, emb_ref, Wl_ref, bl_ref, Wr_ref,
                   out_ref, rinv_ref):
  agg = agg_ref[0] + agg_ref[1]
  cnt = cnt_ref[0] + cnt_ref[1]
  rinv = 1.0 / jnp.maximum(cnt, 1.0)
  mean = agg * rinv
  u = jnp.dot(emb_ref[...], Wr_ref[...], precision=_PREC)
  t = jnp.dot(mean, Wl_ref[...], precision=_PREC) + bl_ref[...] + u
  out_ref[...] = jnp.maximum(_l2norm(t), 0.0)
  rinv_ref[...] = rinv


def _tc_react(aggP, cnt_col, emb_reaction, Wl, bl, Wr):
  grid = (R_PAD // 1024,)
  return pl.pallas_call(
      _tc_react_body,
      grid=grid,
      in_specs=[
          pl.BlockSpec((NC, 1024, D), lambda i: (0, i, 0)),
          pl.BlockSpec((NC, 1024, 1), lambda i: (0, i, 0)),
          pl.BlockSpec((1, D), lambda i: (0, 0)),
          pl.BlockSpec((D, D), lambda i: (0, 0)),
          pl.BlockSpec((D,), lambda i: (0,)),
          pl.BlockSpec((D, D), lambda i: (0, 0)),
      ],
      out_specs=[
          pl.BlockSpec((1024, D), lambda i: (i, 0)),
          pl.BlockSpec((1024, 1), lambda i: (i, 0)),
      ],
      out_shape=[
          jax.ShapeDtypeStruct((R_PAD, D), jnp.float32),
          jax.ShapeDtypeStruct((R_PAD, 1), jnp.float32),
      ],
  )(aggP, cnt_col, emb_reaction, Wl, bl, Wr)


def _tc_final_body(bgg_ref, rinv_ref, hr1_ref, Wl_ref, bl_ref, Wr_ref,
                   Wo_ref, bo_ref, out_ref):
  agg = bgg_ref[0] + bgg_ref[1]
  mean = agg * rinv_ref[...]
  t = (jnp.dot(mean, Wl_ref[...], precision=_PREC) + bl_ref[...]
       + jnp.dot(hr1_ref[...], Wr_ref[...], precision=_PREC))
  h = jnp.maximum(_l2norm(t), 0.0)
  out_ref[...] = jnp.dot(h, Wo_ref[...], precision=_PREC) + bo_ref[...]


def _tc_final(bggP, rinv, h_r1, Wl, bl, Wr, W_out, b_out):
  grid = (R_PAD // 1024,)
  return pl.pallas_call(
      _tc_final_body,
      grid=grid,
      in_specs=[
          pl.BlockSpec((NC, 1024, D), lambda i: (0, i, 0)),
          pl.BlockSpec((1024, 1), lambda i: (i, 0)),
          pl.BlockSpec((1024, D), lambda i: (i, 0)),
          pl.BlockSpec((D, D), lambda i: (0, 0)),
          pl.BlockSpec((D,), lambda i: (0,)),
          pl.BlockSpec((D, D), lambda i: (0, 0)),
          pl.BlockSpec((D, OUT), lambda i: (0, 0)),
          pl.BlockSpec((OUT,), lambda i: (0,)),
      ],
      out_specs=pl.BlockSpec((1024, OUT), lambda i: (i, 0)),
      out_shape=jax.ShapeDtypeStruct((R_PAD, OUT), jnp.float32),
  )(bggP, rinv, h_r1, Wl, bl, Wr, W_out, b_out)


def kernel(x_reaction, x_protein, edge_index_pr, edge_index_rp, emb_reaction,
           emb_protein, Wl_pr_0, bl_pr_0, Wr_pr_0, Wl_rp_0, bl_rp_0, Wr_rp_0,
           Wl_pr_1, bl_pr_1, Wr_pr_1, Wl_rp_1, bl_rp_1, Wr_rp_1, W_out, b_out):
  del x_reaction, Wl_rp_1, bl_rp_1, Wr_rp_1
  xp_pad = jnp.pad(x_protein[:, 0], (0, P_PAD - N_P))
  src_pr = edge_index_pr[0]
  dst_pr = edge_index_pr[1]
  dst_rp = edge_index_rp[1]
  ones_tab = jnp.ones((P_PAD, D), jnp.float32)

  hp0 = _sc_hp0(xp_pad, emb_protein)
  aggP = _sc_edge_agg(hp0, src_pr, dst_pr).reshape(NC, R_PAD, D)
  cntP = _sc_edge_agg(ones_tab, src_pr, dst_pr).reshape(NC, R_PAD, D)
  cnt_col = cntP[:, :, 0:1]
  flag = _sc_flags(dst_rp)

  h_p1 = _tc_protein(hp0, flag, emb_reaction, Wl_rp_0, bl_rp_0, Wr_rp_0)
  h_r1, rinv = _tc_react(aggP, cnt_col, emb_reaction, Wl_pr_0, bl_pr_0, Wr_pr_0)

  bggP = _sc_edge_agg(h_p1, src_pr, dst_pr).reshape(NC, R_PAD, D)
  out_pad = _tc_final(bggP, rinv, h_r1, Wl_pr_1, bl_pr_1, Wr_pr_1, W_out, b_out)
  return out_pad[:N_R]


# speedup vs baseline: 5.4230x; 1.2022x over previous
import functools

import jax
import jax.numpy as jnp
from jax import lax
from jax.experimental import pallas as pl
from jax.experimental.pallas import tpu as pltpu
from jax.experimental.pallas import tpu_sc as plsc

N_R = 10000
N_P = 50000
E = 320000
D = 128
OUT = 2

NC = 2
NS = 16
NW = NC * NS

R_PAD = 10240
P_PAD = 50176
K = 128

ECH = E // K
ECH_Q, ECH_R = divmod(ECH, NW)
KH = 64
PCH = P_PAD // KH
PCH_Q, PCH_R = divmod(PCH, NW)

FW = P_PAD // NW
SCH = 2048
SCH_N, SCH_T = divmod(E, SCH)

_MESH = plsc.VectorSubcoreMesh(
    core_axis_name="c", subcore_axis_name="s", num_cores=NC, num_subcores=NS)


def _worker_id():
  return lax.axis_index("s") * NC + lax.axis_index("c")


def _sc_hp0(xp_pad, emb_aug):
  @functools.partial(
      pl.kernel,
      out_type=jax.ShapeDtypeStruct((P_PAD, D), jnp.float32),
      mesh=_MESH,
      compiler_params=pltpu.CompilerParams(needs_layout_passes=False),
      scratch_types=[
          pltpu.VMEM((KH,), jnp.int32),
          pltpu.VMEM((KH, D), jnp.float32),
          pltpu.SemaphoreType.DMA,
      ],
  )
  def body(xp_hbm, emb_hbm, hp0_hbm, sidx, rows, sem):
    gw = _worker_id()
    nhp = PCH_Q + jnp.where(gw < PCH_R, 1, 0)

    @pl.loop(0, nhp)
    def _hp(i):
      off = (gw + i * NW) * KH
      pltpu.sync_copy(xp_hbm.at[pl.ds(off, KH)], sidx)
      pltpu.async_copy(emb_hbm.at[sidx], rows, sem).wait()
      pltpu.sync_copy(rows, hp0_hbm.at[pl.ds(off, KH)])

  return body(xp_pad, emb_aug)


def _sc_edge_agg(table, src, dst):
  @functools.partial(
      pl.kernel,
      out_type=jax.ShapeDtypeStruct((NC * R_PAD, D), jnp.float32),
      mesh=_MESH,
      compiler_params=pltpu.CompilerParams(needs_layout_passes=False),
      scratch_types=[
          pltpu.VMEM_SHARED((R_PAD, D), jnp.float32),
          pltpu.VMEM((K,), jnp.int32),
          pltpu.VMEM((K,), jnp.int32),
          pltpu.VMEM((K, D), jnp.float32),
          pltpu.SemaphoreType.DMA,
      ],
  )
  def body(tab_hbm, src_hbm, dst_hbm, agg_hbm, agg_s, sidx, didx, rows, sem):
    c = lax.axis_index("c")
    s = lax.axis_index("s")
    gw = _worker_id()

    @pl.loop(0, K)
    def _z(r):
      for j in range(D // 16):
        rows[r, pl.ds(j * 16, 16)] = jnp.zeros((16,), jnp.float32)

    for t in range(5):
      pltpu.sync_copy(rows, agg_s.at[pl.ds(s * 640 + t * K, K)])
    plsc.subcore_barrier()

    nch = ECH_Q + jnp.where(gw < ECH_R, 1, 0)

    @pl.loop(0, nch)
    def _edge(i):
      off = (gw + i * NW) * K
      pltpu.sync_copy(src_hbm.at[pl.ds(off, K)], sidx)
      pltpu.sync_copy(dst_hbm.at[pl.ds(off, K)], didx)
      pltpu.async_copy(tab_hbm.at[sidx], rows, sem).wait()
      pltpu.sync_copy(rows, agg_s.at[didx], add=True)

    plsc.subcore_barrier()
    for t in range(5):
      pltpu.sync_copy(agg_s.at[pl.ds(s * 640 + t * K, K)], rows)
      pltpu.sync_copy(rows, agg_hbm.at[pl.ds(c * R_PAD + s * 640 + t * K, K)])

  return body(table, src, dst)


CG = 8
CRNG = R_PAD // (NW // CG)
CEDG = E // CG
CCH_N, CCH_T = divmod(CEDG, SCH)


def _sc_flags(dst_pr, dst_rp):
  @functools.partial(
      pl.kernel,
      out_type=(
          jax.ShapeDtypeStruct((P_PAD,), jnp.float32),
          jax.ShapeDtypeStruct((CG, R_PAD), jnp.float32),
      ),
      mesh=_MESH,
      compiler_params=pltpu.CompilerParams(needs_layout_passes=False),
      scratch_types=[
          pltpu.VMEM((SCH,), jnp.int32),
          pltpu.VMEM((FW + 16,), jnp.float32),
          pltpu.VMEM((CRNG + 16,), jnp.float32),
      ],
  )
  def body(dpr_hbm, drp_hbm, flag_hbm, cnt_hbm, dbuf, hist, chist):
    gw = _worker_id()

    @pl.loop(0, (FW + 16) // 16)
    def _zh(r):
      hist[pl.ds(r * 16, 16)] = jnp.zeros((16,), jnp.float32)

    @pl.loop(0, (CRNG + 16) // 16)
    def _zc(r):
      chist[pl.ds(r * 16, 16)] = jnp.zeros((16,), jnp.float32)

    base = gw * FW

    def _mark(nvec):
      def mark(j):
        v = dbuf[pl.ds(j * 16, 16)]
        m = (v >= base) & (v < base + FW)
        idx = jnp.where(m, v - base, FW)
        plsc.store_scatter(hist, [idx], jnp.full((16,), 1.0, jnp.float32))
      return pl.loop(0, nvec)(mark)

    @pl.loop(0, SCH_N)
    def _scan(i):
      pltpu.sync_copy(drp_hbm.at[pl.ds(i * SCH, SCH)], dbuf)
      _mark(SCH // 16)

    pltpu.sync_copy(drp_hbm.at[pl.ds(SCH_N * SCH, SCH_T)], dbuf.at[pl.ds(0, SCH_T)])
    _mark(SCH_T // 16)

    pltpu.sync_copy(hist.at[pl.ds(0, FW)], flag_hbm.at[pl.ds(base, FW)])

    grp = gw // (NW // CG)
    cbase = (gw % (NW // CG)) * CRNG
    ebase = grp * CEDG

    def _count(nvec):
      def count(j):
        v = dbuf[pl.ds(j * 16, 16)]
        m = (v >= cbase) & (v < cbase + CRNG)
        idx = jnp.where(m, v - cbase, CRNG)
        cnt, mlast = plsc.scan_count(v, mask=m)
        plsc.addupdate_scatter(chist, [idx], cnt.astype(jnp.float32),
                               mask=mlast & m)
      return pl.loop(0, nvec)(count)

    @pl.loop(0, CCH_N)
    def _cscan(i):
      pltpu.sync_copy(dpr_hbm.at[pl.ds(ebase + i * SCH, SCH)], dbuf)
      _count(SCH // 16)

    pltpu.sync_copy(dpr_hbm.at[pl.ds(ebase + CCH_N * SCH, CCH_T)],
                    dbuf.at[pl.ds(0, CCH_T)])
    _count(CCH_T // 16)

    pltpu.sync_copy(chist.at[pl.ds(0, CRNG)],
                    cnt_hbm.at[grp, pl.ds(cbase, CRNG)])

  return body(dst_pr, dst_rp)


_PREC = lax.Precision.HIGHEST


def _l2norm(t):
  return t / jnp.maximum(jnp.sqrt(jnp.sum(t * t, axis=-1, keepdims=True)), 1e-12)


def _tc_protein_body(hp0_ref, flag_ref, emb_ref, Wl_ref, bl_ref, Wr_ref, out_ref):
  flag = lax.broadcast_in_dim(flag_ref[...], (1024, D), (0,))
  u = jnp.dot(emb_ref[...], Wl_ref[...], precision=_PREC)
  t = (flag * u + bl_ref[...]
       + jnp.dot(hp0_ref[...], Wr_ref[...], precision=_PREC))
  out_ref[...] = jnp.maximum(_l2norm(t), 0.0)


def _tc_protein(hp0, flag, emb_reaction, Wl, bl, Wr):
  grid = (P_PAD // 1024,)
  return pl.pallas_call(
      _tc_protein_body,
      grid=grid,
      in_specs=[
          pl.BlockSpec((1024, D), lambda i: (i, 0)),
          pl.BlockSpec((1024,), lambda i: (i,)),
          pl.BlockSpec((1, D), lambda i: (0, 0)),
          pl.BlockSpec((D, D), lambda i: (0, 0)),
          pl.BlockSpec((D,), lambda i: (0,)),
          pl.BlockSpec((D, D), lambda i: (0, 0)),
      ],
      out_specs=pl.BlockSpec((1024, D), lambda i: (i, 0)),
      out_shape=jax.ShapeDtypeStruct((P_PAD, D), jnp.float32),
  )(hp0, flag, emb_reaction, Wl, bl, Wr)


def _tc_react_body(agg_ref, cnt_ref, emb_ref, Wl_ref, bl_ref, Wr_ref,
                   out_ref, rinv_ref):
  agg = agg_ref[0] + agg_ref[1]
  cnt = jnp.sum(cnt_ref[...], axis=0)
  rinv1 = 1.0 / jnp.maximum(cnt, 1.0)
  rinv = lax.broadcast_in_dim(rinv1, (1024, 1), (0,))
  mean = agg * lax.broadcast_in_dim(rinv1, (1024, D), (0,))
  u = jnp.dot(emb_ref[...], Wr_ref[...], precision=_PREC)
  t = jnp.dot(mean, Wl_ref[...], precision=_PREC) + bl_ref[...] + u
  out_ref[...] = jnp.maximum(_l2norm(t), 0.0)
  rinv_ref[...] = rinv


def _tc_react(aggP, cnt_col, emb_reaction, Wl, bl, Wr):
  grid = (R_PAD // 1024,)
  return pl.pallas_call(
      _tc_react_body,
      grid=grid,
      in_specs=[
          pl.BlockSpec((NC, 1024, D), lambda i: (0, i, 0)),
          pl.BlockSpec((CG, 1024), lambda i: (0, i)),
          pl.BlockSpec((1, D), lambda i: (0, 0)),
          pl.BlockSpec((D, D), lambda i: (0, 0)),
          pl.BlockSpec((D,), lambda i: (0,)),
          pl.BlockSpec((D, D), lambda i: (0, 0)),
      ],
      out_specs=[
          pl.BlockSpec((1024, D), lambda i: (i, 0)),
          pl.BlockSpec((1024, 1), lambda i: (i, 0)),
      ],
      out_shape=[
          jax.ShapeDtypeStruct((R_PAD, D), jnp.float32),
          jax.ShapeDtypeStruct((R_PAD, 1), jnp.float32),
      ],
  )(aggP, cnt_col, emb_reaction, Wl, bl, Wr)


def _tc_final_body(bgg_ref, rinv_ref, hr1_ref, Wl_ref, bl_ref, Wr_ref,
                   Wo_ref, bo_ref, out_ref):
  agg = bgg_ref[0] + bgg_ref[1]
  mean = agg * rinv_ref[...]
  t = (jnp.dot(mean, Wl_ref[...], precision=_PREC) + bl_ref[...]
       + jnp.dot(hr1_ref[...], Wr_ref[...], precision=_PREC))
  h = jnp.maximum(_l2norm(t), 0.0)
  out_ref[...] = jnp.dot(h, Wo_ref[...], precision=_PREC) + bo_ref[...]


def _tc_final(bggP, rinv, h_r1, Wl, bl, Wr, W_out, b_out):
  grid = (R_PAD // 1024,)
  return pl.pallas_call(
      _tc_final_body,
      grid=grid,
      in_specs=[
          pl.BlockSpec((NC, 1024, D), lambda i: (0, i, 0)),
          pl.BlockSpec((1024, 1), lambda i: (i, 0)),
          pl.BlockSpec((1024, D), lambda i: (i, 0)),
          pl.BlockSpec((D, D), lambda i: (0, 0)),
          pl.BlockSpec((D,), lambda i: (0,)),
          pl.BlockSpec((D, D), lambda i: (0, 0)),
          pl.BlockSpec((D, OUT), lambda i: (0, 0)),
          pl.BlockSpec((OUT,), lambda i: (0,)),
      ],
      out_specs=pl.BlockSpec((1024, OUT), lambda i: (i, 0)),
      out_shape=jax.ShapeDtypeStruct((R_PAD, OUT), jnp.float32),
  )(bggP, rinv, h_r1, Wl, bl, Wr, W_out, b_out)


def kernel(x_reaction, x_protein, edge_index_pr, edge_index_rp, emb_reaction,
           emb_protein, Wl_pr_0, bl_pr_0, Wr_pr_0, Wl_rp_0, bl_rp_0, Wr_rp_0,
           Wl_pr_1, bl_pr_1, Wr_pr_1, Wl_rp_1, bl_rp_1, Wr_rp_1, W_out, b_out):
  del x_reaction, Wl_rp_1, bl_rp_1, Wr_rp_1
  xp_pad = jnp.pad(x_protein[:, 0], (0, P_PAD - N_P))
  src_pr = edge_index_pr[0]
  dst_pr = edge_index_pr[1]
  dst_rp = edge_index_rp[1]
  hp0 = _sc_hp0(xp_pad, emb_protein)
  aggP = _sc_edge_agg(hp0, src_pr, dst_pr).reshape(NC, R_PAD, D)
  flag, cnt8 = _sc_flags(dst_pr, dst_rp)

  h_p1 = _tc_protein(hp0, flag, emb_reaction, Wl_rp_0, bl_rp_0, Wr_rp_0)
  h_r1, rinv = _tc_react(aggP, cnt8, emb_reaction, Wl_pr_0, bl_pr_0, Wr_pr_0)

  bggP = _sc_edge_agg(h_p1, src_pr, dst_pr).reshape(NC, R_PAD, D)
  out_pad = _tc_final(bggP, rinv, h_r1, Wl_pr_1, bl_pr_1, Wr_pr_1, W_out, b_out)
  return out_pad[:N_R]


# speedup vs baseline: 6.0420x; 1.1141x over previous
import functools

import jax
import jax.numpy as jnp
from jax import lax
from jax.experimental import pallas as pl
from jax.experimental.pallas import tpu as pltpu
from jax.experimental.pallas import tpu_sc as plsc

N_R = 10000
N_P = 50000
E = 320000
D = 128
OUT = 2

NC = 2
NS = 16
NW = NC * NS

R_PAD = 10240
P_PAD = 50176
K = 128

ECH = E // K
ECH_Q, ECH_R = divmod(ECH, NW)
KH = 64
PCH = P_PAD // KH
PCH_Q, PCH_R = divmod(PCH, NW)

FW = P_PAD // NW
SCH = 2048
SCH_N, SCH_T = divmod(E, SCH)

_MESH = plsc.VectorSubcoreMesh(
    core_axis_name="c", subcore_axis_name="s", num_cores=NC, num_subcores=NS)


def _worker_id():
  return lax.axis_index("s") * NC + lax.axis_index("c")


def _sc_hp0(xp_pad, emb_aug):
  @functools.partial(
      pl.kernel,
      out_type=jax.ShapeDtypeStruct((P_PAD, D), jnp.float32),
      mesh=_MESH,
      compiler_params=pltpu.CompilerParams(needs_layout_passes=False),
      scratch_types=[
          pltpu.VMEM((KH,), jnp.int32),
          pltpu.VMEM((KH, D), jnp.float32),
          pltpu.SemaphoreType.DMA,
      ],
  )
  def body(xp_hbm, emb_hbm, hp0_hbm, sidx, rows, sem):
    gw = _worker_id()
    nhp = PCH_Q + jnp.where(gw < PCH_R, 1, 0)

    @pl.loop(0, nhp)
    def _hp(i):
      off = (gw + i * NW) * KH
      pltpu.sync_copy(xp_hbm.at[pl.ds(off, KH)], sidx)
      pltpu.async_copy(emb_hbm.at[sidx], rows, sem).wait()
      pltpu.sync_copy(rows, hp0_hbm.at[pl.ds(off, KH)])

  return body(xp_pad, emb_aug)


def _sc_edge_agg(table, src, dst):
  @functools.partial(
      pl.kernel,
      out_type=jax.ShapeDtypeStruct((NC * R_PAD, D), jnp.float32),
      mesh=_MESH,
      compiler_params=pltpu.CompilerParams(needs_layout_passes=False),
      scratch_types=[
          pltpu.VMEM_SHARED((R_PAD, D), jnp.float32),
          pltpu.VMEM((K,), jnp.int32),
          pltpu.VMEM((K,), jnp.int32),
          pltpu.VMEM((K, D), jnp.float32),
          pltpu.VMEM((K,), jnp.int32),
          pltpu.VMEM((K,), jnp.int32),
          pltpu.VMEM((K, D), jnp.float32),
          pltpu.SemaphoreType.DMA,
          pltpu.SemaphoreType.DMA,
          pltpu.SemaphoreType.DMA,
      ],
  )
  def body(tab_hbm, src_hbm, dst_hbm, agg_hbm, agg_s,
           sidx0, didx0, rows0, sidx1, didx1, rows1, semg, sems0, sems1):
    c = lax.axis_index("c")
    s = lax.axis_index("s")
    gw = _worker_id()
    sidx = (sidx0, sidx1)
    didx = (didx0, didx1)
    rows = (rows0, rows1)
    sems = (sems0, sems1)

    @pl.loop(0, K)
    def _z(r):
      for j in range(D // 16):
        rows0[r, pl.ds(j * 16, 16)] = jnp.zeros((16,), jnp.float32)

    for t in range(5):
      pltpu.sync_copy(rows0, agg_s.at[pl.ds(s * 640 + t * K, K)])
    plsc.subcore_barrier()

    extra = gw < ECH_R

    def _chunk(j, b, first):
      if not first:
        pltpu.make_async_copy(rows[b], agg_s.at[didx[b]], sems[b]).wait()
      off = (gw + j * NW) * K
      pltpu.sync_copy(src_hbm.at[pl.ds(off, K)], sidx[b])
      pltpu.sync_copy(dst_hbm.at[pl.ds(off, K)], didx[b])
      pltpu.async_copy(tab_hbm.at[sidx[b]], rows[b], semg).wait()
      pltpu.async_copy(rows[b], agg_s.at[didx[b]], sems[b], add=True)

    _chunk(0, 0, True)
    _chunk(1, 1, True)

    @pl.loop(1, ECH_Q // 2)
    def _pairs(i):
      _chunk(2 * i, 0, False)
      _chunk(2 * i + 1, 1, False)

    pltpu.make_async_copy(rows0, agg_s.at[didx0], sems0).wait()

    @pl.when(extra)
    def _():
      _chunk(ECH_Q, 0, True)

    pltpu.make_async_copy(rows1, agg_s.at[didx1], sems1).wait()

    @pl.when(extra)
    def _():
      pltpu.make_async_copy(rows0, agg_s.at[didx0], sems0).wait()

    plsc.subcore_barrier()
    for t in range(5):
      pltpu.sync_copy(agg_s.at[pl.ds(s * 640 + t * K, K)], rows0)
      pltpu.sync_copy(rows0, agg_hbm.at[pl.ds(c * R_PAD + s * 640 + t * K, K)])

  return body(table, src, dst)


CG = 8
CRNG = R_PAD // (NW // CG)
CEDG = E // CG
CCH_N, CCH_T = divmod(CEDG, SCH)


def _sc_flags(dst_pr, dst_rp):
  @functools.partial(
      pl.kernel,
      out_type=(
          jax.ShapeDtypeStruct((P_PAD,), jnp.float32),
          jax.ShapeDtypeStruct((CG, R_PAD), jnp.float32),
      ),
      mesh=_MESH,
      compiler_params=pltpu.CompilerParams(needs_layout_passes=False),
      scratch_types=[
          pltpu.VMEM((SCH,), jnp.int32),
          pltpu.VMEM((FW + 16,), jnp.float32),
          pltpu.VMEM((CRNG + 16,), jnp.float32),
      ],
  )
  def body(dpr_hbm, drp_hbm, flag_hbm, cnt_hbm, dbuf, hist, chist):
    gw = _worker_id()

    @pl.loop(0, (FW + 16) // 16)
    def _zh(r):
      hist[pl.ds(r * 16, 16)] = jnp.zeros((16,), jnp.float32)

    @pl.loop(0, (CRNG + 16) // 16)
    def _zc(r):
      chist[pl.ds(r * 16, 16)] = jnp.zeros((16,), jnp.float32)

    base = gw * FW

    def _mark(nvec):
      def mark(j):
        v = dbuf[pl.ds(j * 16, 16)]
        m = (v >= base) & (v < base + FW)
        idx = jnp.where(m, v - base, FW)
        plsc.store_scatter(hist, [idx], jnp.full((16,), 1.0, jnp.float32))
      return pl.loop(0, nvec)(mark)

    @pl.loop(0, SCH_N)
    def _scan(i):
      pltpu.sync_copy(drp_hbm.at[pl.ds(i * SCH, SCH)], dbuf)
      _mark(SCH // 16)

    pltpu.sync_copy(drp_hbm.at[pl.ds(SCH_N * SCH, SCH_T)], dbuf.at[pl.ds(0, SCH_T)])
    _mark(SCH_T // 16)

    pltpu.sync_copy(hist.at[pl.ds(0, FW)], flag_hbm.at[pl.ds(base, FW)])

    grp = gw // (NW // CG)
    cbase = (gw % (NW // CG)) * CRNG
    ebase = grp * CEDG

    def _count(nvec):
      def count(j):
        v = dbuf[pl.ds(j * 16, 16)]
        m = (v >= cbase) & (v < cbase + CRNG)
        idx = jnp.where(m, v - cbase, CRNG)
        cnt, mlast = plsc.scan_count(v, mask=m)
        plsc.addupdate_scatter(chist, [idx], cnt.astype(jnp.float32),
                               mask=mlast & m)
      return pl.loop(0, nvec)(count)

    @pl.loop(0, CCH_N)
    def _cscan(i):
      pltpu.sync_copy(dpr_hbm.at[pl.ds(ebase + i * SCH, SCH)], dbuf)
      _count(SCH // 16)

    pltpu.sync_copy(dpr_hbm.at[pl.ds(ebase + CCH_N * SCH, CCH_T)],
                    dbuf.at[pl.ds(0, CCH_T)])
    _count(CCH_T // 16)

    pltpu.sync_copy(chist.at[pl.ds(0, CRNG)],
                    cnt_hbm.at[grp, pl.ds(cbase, CRNG)])

  return body(dst_pr, dst_rp)


_PREC = lax.Precision.HIGHEST


def _l2norm(t):
  return t / jnp.maximum(jnp.sqrt(jnp.sum(t * t, axis=-1, keepdims=True)), 1e-12)


def _tc_protein_body(hp0_ref, flag_ref, emb_ref, Wl_ref, bl_ref, Wr_ref, out_ref):
  flag = lax.broadcast_in_dim(flag_ref[...], (1024, D), (0,))
  u = jnp.dot(emb_ref[...], Wl_ref[...], precision=_PREC)
  t = (flag * u + bl_ref[...]
       + jnp.dot(hp0_ref[...], Wr_ref[...], precision=_PREC))
  out_ref[...] = jnp.maximum(_l2norm(t), 0.0)


def _tc_protein(hp0, flag, emb_reaction, Wl, bl, Wr):
  grid = (P_PAD // 1024,)
  return pl.pallas_call(
      _tc_protein_body,
      grid=grid,
      in_specs=[
          pl.BlockSpec((1024, D), lambda i: (i, 0)),
          pl.BlockSpec((1024,), lambda i: (i,)),
          pl.BlockSpec((1, D), lambda i: (0, 0)),
          pl.BlockSpec((D, D), lambda i: (0, 0)),
          pl.BlockSpec((D,), lambda i: (0,)),
          pl.BlockSpec((D, D), lambda i: (0, 0)),
      ],
      out_specs=pl.BlockSpec((1024, D), lambda i: (i, 0)),
      out_shape=jax.ShapeDtypeStruct((P_PAD, D), jnp.float32),
  )(hp0, flag, emb_reaction, Wl, bl, Wr)


def _tc_react_body(agg_ref, cnt_ref, emb_ref, Wl_ref, bl_ref, Wr_ref,
                   out_ref, rinv_ref):
  agg = agg_ref[0] + agg_ref[1]
  cnt = jnp.sum(cnt_ref[...], axis=0)
  rinv1 = 1.0 / jnp.maximum(cnt, 1.0)
  rinv = lax.broadcast_in_dim(rinv1, (1024, 1), (0,))
  mean = agg * lax.broadcast_in_dim(rinv1, (1024, D), (0,))
  u = jnp.dot(emb_ref[...], Wr_ref[...], precision=_PREC)
  t = jnp.dot(mean, Wl_ref[...], precision=_PREC) + bl_ref[...] + u
  out_ref[...] = jnp.maximum(_l2norm(t), 0.0)
  rinv_ref[...] = rinv


def _tc_react(aggP, cnt_col, emb_reaction, Wl, bl, Wr):
  grid = (R_PAD // 1024,)
  return pl.pallas_call(
      _tc_react_body,
      grid=grid,
      in_specs=[
          pl.BlockSpec((NC, 1024, D), lambda i: (0, i, 0)),
          pl.BlockSpec((CG, 1024), lambda i: (0, i)),
          pl.BlockSpec((1, D), lambda i: (0, 0)),
          pl.BlockSpec((D, D), lambda i: (0, 0)),
          pl.BlockSpec((D,), lambda i: (0,)),
          pl.BlockSpec((D, D), lambda i: (0, 0)),
      ],
      out_specs=[
          pl.BlockSpec((1024, D), lambda i: (i, 0)),
          pl.BlockSpec((1024, 1), lambda i: (i, 0)),
      ],
      out_shape=[
          jax.ShapeDtypeStruct((R_PAD, D), jnp.float32),
          jax.ShapeDtypeStruct((R_PAD, 1), jnp.float32),
      ],
  )(aggP, cnt_col, emb_reaction, Wl, bl, Wr)


def _tc_final_body(bgg_ref, rinv_ref, hr1_ref, Wl_ref, bl_ref, Wr_ref,
                   Wo_ref, bo_ref, out_ref):
  agg = bgg_ref[0] + bgg_ref[1]
  mean = agg * rinv_ref[...]
  t = (jnp.dot(mean, Wl_ref[...], precision=_PREC) + bl_ref[...]
       + jnp.dot(hr1_ref[...], Wr_ref[...], precision=_PREC))
  h = jnp.maximum(_l2norm(t), 0.0)
  out_ref[...] = jnp.dot(h, Wo_ref[...], precision=_PREC) + bo_ref[...]


def _tc_final(bggP, rinv, h_r1, Wl, bl, Wr, W_out, b_out):
  grid = (R_PAD // 1024,)
  return pl.pallas_call(
      _tc_final_body,
      grid=grid,
      in_specs=[
          pl.BlockSpec((NC, 1024, D), lambda i: (0, i, 0)),
          pl.BlockSpec((1024, 1), lambda i: (i, 0)),
          pl.BlockSpec((1024, D), lambda i: (i, 0)),
          pl.BlockSpec((D, D), lambda i: (0, 0)),
          pl.BlockSpec((D,), lambda i: (0,)),
          pl.BlockSpec((D, D), lambda i: (0, 0)),
          pl.BlockSpec((D, OUT), lambda i: (0, 0)),
          pl.BlockSpec((OUT,), lambda i: (0,)),
      ],
      out_specs=pl.BlockSpec((1024, OUT), lambda i: (i, 0)),
      out_shape=jax.ShapeDtypeStruct((R_PAD, OUT), jnp.float32),
  )(bggP, rinv, h_r1, Wl, bl, Wr, W_out, b_out)


def kernel(x_reaction, x_protein, edge_index_pr, edge_index_rp, emb_reaction,
           emb_protein, Wl_pr_0, bl_pr_0, Wr_pr_0, Wl_rp_0, bl_rp_0, Wr_rp_0,
           Wl_pr_1, bl_pr_1, Wr_pr_1, Wl_rp_1, bl_rp_1, Wr_rp_1, W_out, b_out):
  del x_reaction, Wl_rp_1, bl_rp_1, Wr_rp_1
  xp_pad = jnp.pad(x_protein[:, 0], (0, P_PAD - N_P))
  src_pr = edge_index_pr[0]
  dst_pr = edge_index_pr[1]
  dst_rp = edge_index_rp[1]
  hp0 = _sc_hp0(xp_pad, emb_protein)
  aggP = _sc_edge_agg(hp0, src_pr, dst_pr).reshape(NC, R_PAD, D)
  flag, cnt8 = _sc_flags(dst_pr, dst_rp)

  h_p1 = _tc_protein(hp0, flag, emb_reaction, Wl_rp_0, bl_rp_0, Wr_rp_0)
  h_r1, rinv = _tc_react(aggP, cnt8, emb_reaction, Wl_pr_0, bl_pr_0, Wr_pr_0)

  bggP = _sc_edge_agg(h_p1, src_pr, dst_pr).reshape(NC, R_PAD, D)
  out_pad = _tc_final(bggP, rinv, h_r1, Wl_pr_1, bl_pr_1, Wr_pr_1, W_out, b_out)
  return out_pad[:N_R]


# speedup vs baseline: 9.2166x; 1.5254x over previous
import functools

import jax
import jax.numpy as jnp
from jax import lax
from jax.experimental import pallas as pl
from jax.experimental.pallas import tpu as pltpu
from jax.experimental.pallas import tpu_sc as plsc

N_R = 10000
N_P = 50000
E = 320000
D = 128
OUT = 2

NC = 2
NS = 16
NW = NC * NS

R_PAD = 10240
P_PAD = 50176
K = 128

SB = 8
NSUP = 10
E_PAD = NW * NSUP * SB * K
ECH_P = E_PAD // K

KH = 64
PCH = P_PAD // KH
PCH_Q, PCH_R = divmod(PCH, NW)

FG = 2
FW = P_PAD // (NW // FG)
FEDG = E // FG
SCH = 2048
FCH_N, FCH_T = divmod(FEDG, SCH)

CG = 8
CRNG = R_PAD // (NW // CG)
CEDG = E // CG
CCH_N, CCH_T = divmod(CEDG, SCH)

_MESH = plsc.VectorSubcoreMesh(
    core_axis_name="c", subcore_axis_name="s", num_cores=NC, num_subcores=NS)


def _worker_id():
  return lax.axis_index("s") * NC + lax.axis_index("c")


def _sc_hp0(xp_pad, emb_protein):
  @functools.partial(
      pl.kernel,
      out_type=jax.ShapeDtypeStruct((P_PAD, D), jnp.float32),
      mesh=_MESH,
      compiler_params=pltpu.CompilerParams(needs_layout_passes=False),
      scratch_types=[
          pltpu.VMEM((KH,), jnp.int32),
          pltpu.VMEM((KH, D), jnp.float32),
          pltpu.SemaphoreType.DMA,
      ],
  )
  def body(xp_hbm, emb_hbm, hp0_hbm, sidx, rows, sem):
    gw = _worker_id()
    nhp = PCH_Q + jnp.where(gw < PCH_R, 1, 0)

    @pl.loop(0, nhp)
    def _hp(i):
      off = (gw + i * NW) * KH
      pltpu.sync_copy(xp_hbm.at[pl.ds(off, KH)], sidx)
      pltpu.async_copy(emb_hbm.at[sidx], rows, sem).wait()
      pltpu.sync_copy(rows, hp0_hbm.at[pl.ds(off, KH)])

  return body(xp_pad, emb_protein)


def _sc_edge_agg(table, src2d, dst2d):
  @functools.partial(
      pl.kernel,
      out_type=jax.ShapeDtypeStruct((NC * R_PAD, D), jnp.float32),
      mesh=_MESH,
      compiler_params=pltpu.CompilerParams(needs_layout_passes=False),
      scratch_types=[
          pltpu.VMEM_SHARED((R_PAD, D), jnp.float32),
          pltpu.VMEM((SB, K), jnp.int32),
          pltpu.VMEM((SB, K), jnp.int32),
          pltpu.VMEM((K, D), jnp.float32),
          pltpu.VMEM((K, D), jnp.float32),
          pltpu.SemaphoreType.DMA,
          pltpu.SemaphoreType.DMA,
          pltpu.SemaphoreType.DMA,
      ],
  )
  def body(tab_hbm, src_hbm, dst_hbm, agg_hbm, agg_s,
           sbig, dbig, rows0, rows1, semg, sems0, sems1):
    c = lax.axis_index("c")
    s = lax.axis_index("s")
    gw = _worker_id()
    rows = (rows0, rows1)
    sems = (sems0, sems1)

    @pl.loop(0, K)
    def _z(r):
      for j in range(D // 16):
        rows0[r, pl.ds(j * 16, 16)] = jnp.zeros((16,), jnp.float32)

    for t in range(5):
      pltpu.sync_copy(rows0, agg_s.at[pl.ds(s * 640 + t * K, K)])
    plsc.subcore_barrier()

    ch0 = gw * (NSUP * SB)

    @pl.loop(0, NSUP)
    def _sup(sp):
      base = ch0 + sp * SB
      pltpu.sync_copy(src_hbm.at[pl.ds(base, SB)], sbig)
      pltpu.sync_copy(dst_hbm.at[pl.ds(base, SB)], dbig)
      for q in range(SB):
        b = q % 2
        if q >= 2:
          pltpu.make_async_copy(rows[b], agg_s.at[dbig.at[q]], sems[b]).wait()
        pltpu.async_copy(tab_hbm.at[sbig.at[q]], rows[b], semg).wait()
        pltpu.async_copy(rows[b], agg_s.at[dbig.at[q]], sems[b], add=True)
      pltpu.make_async_copy(rows0, agg_s.at[dbig.at[0]], sems0).wait()
      pltpu.make_async_copy(rows1, agg_s.at[dbig.at[0]], sems1).wait()

    plsc.subcore_barrier()
    for t in range(5):
      pltpu.sync_copy(agg_s.at[pl.ds(s * 640 + t * K, K)], rows0)
      pltpu.sync_copy(rows0, agg_hbm.at[pl.ds(c * R_PAD + s * 640 + t * K, K)])

  return body(table, src2d, dst2d)


def _sc_flags(dst_pr, dst_rp):
  @functools.partial(
      pl.kernel,
      out_type=(
          jax.ShapeDtypeStruct((FG * P_PAD,), jnp.float32),
          jax.ShapeDtypeStruct((CG * R_PAD,), jnp.float32),
      ),
      mesh=_MESH,
      compiler_params=pltpu.CompilerParams(needs_layout_passes=False),
      scratch_types=[
          pltpu.VMEM((SCH,), jnp.int32),
          pltpu.VMEM((SCH,), jnp.int32),
          pltpu.VMEM((FW + 16,), jnp.float32),
          pltpu.VMEM((CRNG + 16,), jnp.float32),
          pltpu.SemaphoreType.DMA,
          pltpu.SemaphoreType.DMA,
      ],
  )
  def body(dpr_hbm, drp_hbm, flag_hbm, cnt_hbm,
           dbuf0, dbuf1, hist, chist, semd0, semd1):
    gw = _worker_id()
    dbuf = (dbuf0, dbuf1)
    semd = (semd0, semd1)

    @pl.loop(0, (FW + 16) // 16)
    def _zh(r):
      hist[pl.ds(r * 16, 16)] = jnp.zeros((16,), jnp.float32)

    @pl.loop(0, (CRNG + 16) // 16)
    def _zc(r):
      chist[pl.ds(r * 16, 16)] = jnp.zeros((16,), jnp.float32)

    fg = gw // (NW // FG)
    base = (gw % (NW // FG)) * FW
    ebase = fg * FEDG

    def _mark(b, nvec):
      def mark(j):
        v = dbuf[b][pl.ds(j * 16, 16)]
        m = (v >= base) & (v < base + FW)
        idx = jnp.where(m, v - base, FW)
        plsc.store_scatter(hist, [idx], jnp.full((16,), 1.0, jnp.float32))
      return pl.loop(0, nvec)(mark)

    pltpu.async_copy(drp_hbm.at[pl.ds(ebase, SCH)], dbuf0, semd0)

    @pl.loop(0, FCH_N // 2)
    def _fpair(i):
      pltpu.make_async_copy(drp_hbm.at[pl.ds(ebase, SCH)], dbuf0, semd0).wait()
      pltpu.async_copy(
          drp_hbm.at[pl.ds(ebase + (2 * i + 1) * SCH, SCH)], dbuf1, semd1)
      _mark(0, SCH // 16)
      pltpu.make_async_copy(drp_hbm.at[pl.ds(ebase, SCH)], dbuf1, semd1).wait()

      @pl.when(i < FCH_N // 2 - 1)
      def _():
        pltpu.async_copy(
            drp_hbm.at[pl.ds(ebase + (2 * i + 2) * SCH, SCH)], dbuf0, semd0)
      _mark(1, SCH // 16)

    pltpu.sync_copy(drp_hbm.at[pl.ds(ebase + FCH_N * SCH, FCH_T)],
                    dbuf0.at[pl.ds(0, FCH_T)])
    _mark(0, FCH_T // 16)

    pltpu.sync_copy(hist.at[pl.ds(0, FW)],
                    flag_hbm.at[pl.ds(fg * P_PAD + base, FW)])

    cg = gw // (NW // CG)
    cbase = (gw % (NW // CG)) * CRNG
    cebase = cg * CEDG

    def _count(b, nvec):
      def count(j):
        v = dbuf[b][pl.ds(j * 16, 16)]
        m = (v >= cbase) & (v < cbase + CRNG)
        idx = jnp.where(m, v - cbase, CRNG)
        cnt, mlast = plsc.scan_count(v, mask=m)
        plsc.addupdate_scatter(chist, [idx], cnt.astype(jnp.float32),
                               mask=mlast & m)
      return pl.loop(0, nvec)(count)

    @pl.loop(0, CCH_N)
    def _cscan(i):
      pltpu.sync_copy(dpr_hbm.at[pl.ds(cebase + i * SCH, SCH)], dbuf0)
      _count(0, SCH // 16)

    pltpu.sync_copy(dpr_hbm.at[pl.ds(cebase + CCH_N * SCH, CCH_T)],
                    dbuf0.at[pl.ds(0, CCH_T)])
    _count(0, CCH_T // 16)

    pltpu.sync_copy(chist.at[pl.ds(0, CRNG)],
                    cnt_hbm.at[pl.ds(cg * R_PAD + cbase, CRNG)])

  return body(dst_pr, dst_rp)


_PREC = lax.Precision.HIGHEST
BLK_P = 1024
BLK_R = 1024


def _l2norm(t):
  return t / jnp.maximum(jnp.sqrt(jnp.sum(t * t, axis=-1, keepdims=True)), 1e-12)


def _tc_protein_body(hp0_ref, flag_ref, emb_ref, Wl_ref, bl_ref, Wr_ref, out_ref):
  f1 = jnp.max(flag_ref[...], axis=0)
  flag = lax.broadcast_in_dim(f1, (BLK_P, D), (0,))
  u = jnp.dot(emb_ref[...], Wl_ref[...], precision=_PREC)
  t = (flag * u + bl_ref[...]
       + jnp.dot(hp0_ref[...], Wr_ref[...], precision=_PREC))
  out_ref[...] = jnp.maximum(_l2norm(t), 0.0)


def _tc_protein(hp0, flag, emb_reaction, Wl, bl, Wr):
  grid = (P_PAD // BLK_P,)
  return pl.pallas_call(
      _tc_protein_body,
      grid=grid,
      in_specs=[
          pl.BlockSpec((BLK_P, D), lambda i: (i, 0)),
          pl.BlockSpec((FG, BLK_P), lambda i: (0, i)),
          pl.BlockSpec((1, D), lambda i: (0, 0)),
          pl.BlockSpec((D, D), lambda i: (0, 0)),
          pl.BlockSpec((D,), lambda i: (0,)),
          pl.BlockSpec((D, D), lambda i: (0, 0)),
      ],
      out_specs=pl.BlockSpec((BLK_P, D), lambda i: (i, 0)),
      out_shape=jax.ShapeDtypeStruct((P_PAD, D), jnp.float32),
  )(hp0, flag, emb_reaction, Wl, bl, Wr)


def _tc_react_body(agg_ref, cnt_ref, emb_ref, Wl_ref, bl_ref, Wr_ref,
                   out_ref, rinv_ref):
  agg = agg_ref[0] + agg_ref[1]
  cnt = jnp.sum(cnt_ref[...], axis=0)
  rinv1 = 1.0 / jnp.maximum(cnt, 1.0)
  rinv = lax.broadcast_in_dim(rinv1, (BLK_R, 1), (0,))
  mean = agg * lax.broadcast_in_dim(rinv1, (BLK_R, D), (0,))
  u = jnp.dot(emb_ref[...], Wr_ref[...], precision=_PREC)
  t = jnp.dot(mean, Wl_ref[...], precision=_PREC) + bl_ref[...] + u
  out_ref[...] = jnp.maximum(_l2norm(t), 0.0)
  rinv_ref[...] = rinv


def _tc_react(aggP, cnt8, emb_reaction, Wl, bl, Wr):
  grid = (R_PAD // BLK_R,)
  return pl.pallas_call(
      _tc_react_body,
      grid=grid,
      in_specs=[
          pl.BlockSpec((NC, BLK_R, D), lambda i: (0, i, 0)),
          pl.BlockSpec((CG, BLK_R), lambda i: (0, i)),
          pl.BlockSpec((1, D), lambda i: (0, 0)),
          pl.BlockSpec((D, D), lambda i: (0, 0)),
          pl.BlockSpec((D,), lambda i: (0,)),
          pl.BlockSpec((D, D), lambda i: (0, 0)),
      ],
      out_specs=[
          pl.BlockSpec((BLK_R, D), lambda i: (i, 0)),
          pl.BlockSpec((BLK_R, 1), lambda i: (i, 0)),
      ],
      out_shape=[
          jax.ShapeDtypeStruct((R_PAD, D), jnp.float32),
          jax.ShapeDtypeStruct((R_PAD, 1), jnp.float32),
      ],
  )(aggP, cnt8, emb_reaction, Wl, bl, Wr)


def _tc_final_body(bgg_ref, rinv_ref, hr1_ref, Wl_ref, bl_ref, Wr_ref,
                   Wo_ref, bo_ref, out_ref):
  agg = bgg_ref[0] + bgg_ref[1]
  mean = agg * rinv_ref[...]
  t = (jnp.dot(mean, Wl_ref[...], precision=_PREC) + bl_ref[...]
       + jnp.dot(hr1_ref[...], Wr_ref[...], precision=_PREC))
  h = jnp.maximum(_l2norm(t), 0.0)
  out_ref[...] = jnp.dot(h, Wo_ref[...], precision=_PREC) + bo_ref[...]


def _tc_final(bggP, rinv, h_r1, Wl, bl, Wr, W_out, b_out):
  grid = (R_PAD // BLK_R,)
  return pl.pallas_call(
      _tc_final_body,
      grid=grid,
      in_specs=[
          pl.BlockSpec((NC, BLK_R, D), lambda i: (0, i, 0)),
          pl.BlockSpec((BLK_R, 1), lambda i: (i, 0)),
          pl.BlockSpec((BLK_R, D), lambda i: (i, 0)),
          pl.BlockSpec((D, D), lambda i: (0, 0)),
          pl.BlockSpec((D,), lambda i: (0,)),
          pl.BlockSpec((D, D), lambda i: (0, 0)),
          pl.BlockSpec((D, OUT), lambda i: (0, 0)),
          pl.BlockSpec((OUT,), lambda i: (0,)),
      ],
      out_specs=pl.BlockSpec((BLK_R, OUT), lambda i: (i, 0)),
      out_shape=jax.ShapeDtypeStruct((R_PAD, OUT), jnp.float32),
  )(bggP, rinv, h_r1, Wl, bl, Wr, W_out, b_out)


def kernel(x_reaction, x_protein, edge_index_pr, edge_index_rp, emb_reaction,
           emb_protein, Wl_pr_0, bl_pr_0, Wr_pr_0, Wl_rp_0, bl_rp_0, Wr_rp_0,
           Wl_pr_1, bl_pr_1, Wr_pr_1, Wl_rp_1, bl_rp_1, Wr_rp_1, W_out, b_out):
  del x_reaction, Wl_rp_1, bl_rp_1, Wr_rp_1
  xp_pad = jnp.pad(x_protein[:, 0], (0, P_PAD - N_P))
  src_pr = edge_index_pr[0]
  dst_pr = edge_index_pr[1]
  dst_rp = edge_index_rp[1]
  npad = E_PAD - E
  pad_src = jnp.arange(npad, dtype=jnp.int32) % N_P
  pad_dst = N_R + (jnp.arange(npad, dtype=jnp.int32) % (R_PAD - N_R))
  src2d = jnp.concatenate([src_pr, pad_src]).reshape(ECH_P, K)
  dst2d = jnp.concatenate([dst_pr, pad_dst]).reshape(ECH_P, K)

  hp0 = _sc_hp0(xp_pad, emb_protein)
  aggP = _sc_edge_agg(hp0, src2d, dst2d).reshape(NC, R_PAD, D)
  flag, cnt8 = _sc_flags(dst_pr, dst_rp)
  flag = flag.reshape(FG, P_PAD)
  cnt8 = cnt8.reshape(CG, R_PAD)

  h_p1 = _tc_protein(hp0, flag, emb_reaction, Wl_rp_0, bl_rp_0, Wr_rp_0)
  h_r1, rinv = _tc_react(aggP, cnt8, emb_reaction, Wl_pr_0, bl_pr_0, Wr_pr_0)

  bggP = _sc_edge_agg(h_p1, src2d, dst2d).reshape(NC, R_PAD, D)
  out_pad = _tc_final(bggP, rinv, h_r1, Wl_pr_1, bl_pr_1, Wr_pr_1, W_out, b_out)
  return out_pad[:N_R]


# speedup vs baseline: 9.4488x; 1.0252x over previous
import functools

import jax
import jax.numpy as jnp
from jax import lax
from jax.experimental import pallas as pl
from jax.experimental.pallas import tpu as pltpu
from jax.experimental.pallas import tpu_sc as plsc

N_R = 10000
N_P = 50000
E = 320000
D = 128
OUT = 2

NC = 2
NS = 16
NW = NC * NS

R_PAD = 10240
P_PAD = 50176
K = 128

SB = 8
NSUP = 10
E_PAD = NW * NSUP * SB * K
ECH_P = E_PAD // K

KH = 64
PCH = P_PAD // KH
PCH_Q, PCH_R = divmod(PCH, NW)

FG = 2
FW = P_PAD // (NW // FG)
FEDG = E // FG
SCH = 2048
FCH_N, FCH_T = divmod(FEDG, SCH)

CG = 8
CRNG = R_PAD // (NW // CG)
CEDG = E // CG
CCH_N, CCH_T = divmod(CEDG, SCH)

_MESH = plsc.VectorSubcoreMesh(
    core_axis_name="c", subcore_axis_name="s", num_cores=NC, num_subcores=NS)


def _worker_id():
  return lax.axis_index("s") * NC + lax.axis_index("c")


def _sc_hp0(xp_pad, emb_protein):
  @functools.partial(
      pl.kernel,
      out_type=jax.ShapeDtypeStruct((P_PAD, D), jnp.float32),
      mesh=_MESH,
      compiler_params=pltpu.CompilerParams(needs_layout_passes=False),
      scratch_types=[
          pltpu.VMEM((KH,), jnp.int32),
          pltpu.VMEM((KH, D), jnp.float32),
          pltpu.SemaphoreType.DMA,
      ],
  )
  def body(xp_hbm, emb_hbm, hp0_hbm, sidx, rows, sem):
    gw = _worker_id()
    nhp = PCH_Q + jnp.where(gw < PCH_R, 1, 0)

    @pl.loop(0, nhp)
    def _hp(i):
      off = (gw + i * NW) * KH
      pltpu.sync_copy(xp_hbm.at[pl.ds(off, KH)], sidx)
      pltpu.async_copy(emb_hbm.at[sidx], rows, sem).wait()
      pltpu.sync_copy(rows, hp0_hbm.at[pl.ds(off, KH)])

  return body(xp_pad, emb_protein)


def _sc_edge_agg(table, src2d, dst2d):
  @functools.partial(
      pl.kernel,
      out_type=jax.ShapeDtypeStruct((NC * R_PAD, D), jnp.float32),
      mesh=_MESH,
      compiler_params=pltpu.CompilerParams(needs_layout_passes=False),
      scratch_types=[
          pltpu.VMEM_SHARED((R_PAD, D), jnp.float32),
          pltpu.VMEM((SB, K), jnp.int32),
          pltpu.VMEM((SB, K), jnp.int32),
          pltpu.VMEM((K, D), jnp.float32),
          pltpu.VMEM((K, D), jnp.float32),
          pltpu.SemaphoreType.DMA,
          pltpu.SemaphoreType.DMA,
          pltpu.SemaphoreType.DMA,
      ],
  )
  def body(tab_hbm, src_hbm, dst_hbm, agg_hbm, agg_s,
           sbig, dbig, rows0, rows1, semg, sems0, sems1):
    c = lax.axis_index("c")
    s = lax.axis_index("s")
    gw = _worker_id()
    rows = (rows0, rows1)
    sems = (sems0, sems1)

    @pl.loop(0, K)
    def _z(r):
      for j in range(D // 16):
        rows0[r, pl.ds(j * 16, 16)] = jnp.zeros((16,), jnp.float32)

    for t in range(5):
      pltpu.sync_copy(rows0, agg_s.at[pl.ds(s * 640 + t * K, K)])
    plsc.subcore_barrier()

    ch0 = gw * (NSUP * SB)

    @pl.loop(0, NSUP)
    def _sup(sp):
      base = ch0 + sp * SB
      pltpu.sync_copy(src_hbm.at[pl.ds(base, SB)], sbig)
      pltpu.sync_copy(dst_hbm.at[pl.ds(base, SB)], dbig)
      for q in range(SB):
        b = q % 2
        if q >= 2:
          pltpu.make_async_copy(rows[b], agg_s.at[dbig.at[q]], sems[b]).wait()
        pltpu.async_copy(tab_hbm.at[sbig.at[q]], rows[b], semg).wait()
        pltpu.async_copy(rows[b], agg_s.at[dbig.at[q]], sems[b], add=True)
      pltpu.make_async_copy(rows0, agg_s.at[dbig.at[0]], sems0).wait()
      pltpu.make_async_copy(rows1, agg_s.at[dbig.at[0]], sems1).wait()

    plsc.subcore_barrier()
    for t in range(5):
      pltpu.sync_copy(agg_s.at[pl.ds(s * 640 + t * K, K)], rows0)
      pltpu.sync_copy(rows0, agg_hbm.at[pl.ds(c * R_PAD + s * 640 + t * K, K)])

  return body(table, src2d, dst2d)


def _sc_flags(dst_pr, dst_rp):
  @functools.partial(
      pl.kernel,
      out_type=(
          jax.ShapeDtypeStruct((FG * P_PAD,), jnp.float32),
          jax.ShapeDtypeStruct((CG * R_PAD,), jnp.float32),
      ),
      mesh=_MESH,
      compiler_params=pltpu.CompilerParams(needs_layout_passes=False),
      scratch_types=[
          pltpu.VMEM((SCH,), jnp.int32),
          pltpu.VMEM((SCH,), jnp.int32),
          pltpu.VMEM((FW + 16,), jnp.float32),
          pltpu.VMEM((CRNG + 16,), jnp.float32),
          pltpu.SemaphoreType.DMA,
          pltpu.SemaphoreType.DMA,
      ],
  )
  def body(dpr_hbm, drp_hbm, flag_hbm, cnt_hbm,
           dbuf0, dbuf1, hist, chist, semd0, semd1):
    gw = _worker_id()
    dbuf = (dbuf0, dbuf1)
    semd = (semd0, semd1)

    @pl.loop(0, (FW + 16) // 16)
    def _zh(r):
      hist[pl.ds(r * 16, 16)] = jnp.zeros((16,), jnp.float32)

    @pl.loop(0, (CRNG + 16) // 16)
    def _zc(r):
      chist[pl.ds(r * 16, 16)] = jnp.zeros((16,), jnp.float32)

    fg = gw // (NW // FG)
    base = (gw % (NW // FG)) * FW
    ebase = fg * FEDG

    def _mark(b, nvec):
      def mark(j):
        v = dbuf[b][pl.ds(j * 16, 16)]
        m = (v >= base) & (v < base + FW)
        idx = jnp.where(m, v - base, FW)
        plsc.store_scatter(hist, [idx], jnp.full((16,), 1.0, jnp.float32))
      return pl.loop(0, nvec)(mark)

    pltpu.async_copy(drp_hbm.at[pl.ds(ebase, SCH)], dbuf0, semd0)

    @pl.loop(0, FCH_N // 2)
    def _fpair(i):
      pltpu.make_async_copy(drp_hbm.at[pl.ds(ebase, SCH)], dbuf0, semd0).wait()
      pltpu.async_copy(
          drp_hbm.at[pl.ds(ebase + (2 * i + 1) * SCH, SCH)], dbuf1, semd1)
      _mark(0, SCH // 16)
      pltpu.make_async_copy(drp_hbm.at[pl.ds(ebase, SCH)], dbuf1, semd1).wait()

      @pl.when(i < FCH_N // 2 - 1)
      def _():
        pltpu.async_copy(
            drp_hbm.at[pl.ds(ebase + (2 * i + 2) * SCH, SCH)], dbuf0, semd0)
      _mark(1, SCH // 16)

    pltpu.sync_copy(drp_hbm.at[pl.ds(ebase + FCH_N * SCH, FCH_T)],
                    dbuf0.at[pl.ds(0, FCH_T)])
    _mark(0, FCH_T // 16)

    pltpu.sync_copy(hist.at[pl.ds(0, FW)],
                    flag_hbm.at[pl.ds(fg * P_PAD + base, FW)])

    cg = gw // (NW // CG)
    cbase = (gw % (NW // CG)) * CRNG
    cebase = cg * CEDG

    def _count(b, nvec):
      def count(j):
        v = dbuf[b][pl.ds(j * 16, 16)]
        m = (v >= cbase) & (v < cbase + CRNG)
        idx = jnp.where(m, v - cbase, CRNG)
        cnt, mlast = plsc.scan_count(v, mask=m)
        plsc.addupdate_scatter(chist, [idx], cnt.astype(jnp.float32),
                               mask=mlast & m)
      return pl.loop(0, nvec)(count)

    @pl.loop(0, CCH_N)
    def _cscan(i):
      pltpu.sync_copy(dpr_hbm.at[pl.ds(cebase + i * SCH, SCH)], dbuf0)
      _count(0, SCH // 16)

    pltpu.sync_copy(dpr_hbm.at[pl.ds(cebase + CCH_N * SCH, CCH_T)],
                    dbuf0.at[pl.ds(0, CCH_T)])
    _count(0, CCH_T // 16)

    pltpu.sync_copy(chist.at[pl.ds(0, CRNG)],
                    cnt_hbm.at[pl.ds(cg * R_PAD + cbase, CRNG)])

  return body(dst_pr, dst_rp)


_PREC = lax.Precision.DEFAULT
BLK_P = 1024
BLK_R = 1024


def _l2norm(t):
  return t / jnp.maximum(jnp.sqrt(jnp.sum(t * t, axis=-1, keepdims=True)), 1e-12)


def _tc_protein_body(hp0_ref, flag_ref, emb_ref, Wl_ref, bl_ref, Wr_ref, out_ref):
  f1 = jnp.max(flag_ref[...], axis=0)
  flag = lax.broadcast_in_dim(f1, (BLK_P, D), (0,))
  u = jnp.dot(emb_ref[...], Wl_ref[...], precision=_PREC)
  t = (flag * u + bl_ref[...]
       + jnp.dot(hp0_ref[...], Wr_ref[...], precision=_PREC))
  out_ref[...] = jnp.maximum(_l2norm(t), 0.0)


def _tc_protein(hp0, flag, emb_reaction, Wl, bl, Wr):
  grid = (P_PAD // BLK_P,)
  return pl.pallas_call(
      _tc_protein_body,
      grid=grid,
      in_specs=[
          pl.BlockSpec((BLK_P, D), lambda i: (i, 0)),
          pl.BlockSpec((FG, BLK_P), lambda i: (0, i)),
          pl.BlockSpec((1, D), lambda i: (0, 0)),
          pl.BlockSpec((D, D), lambda i: (0, 0)),
          pl.BlockSpec((D,), lambda i: (0,)),
          pl.BlockSpec((D, D), lambda i: (0, 0)),
      ],
      out_specs=pl.BlockSpec((BLK_P, D), lambda i: (i, 0)),
      out_shape=jax.ShapeDtypeStruct((P_PAD, D), jnp.float32),
  )(hp0, flag, emb_reaction, Wl, bl, Wr)


def _tc_react_body(agg_ref, cnt_ref, emb_ref, Wl_ref, bl_ref, Wr_ref,
                   out_ref, rinv_ref):
  agg = agg_ref[0] + agg_ref[1]
  cnt = jnp.sum(cnt_ref[...], axis=0)
  rinv1 = 1.0 / jnp.maximum(cnt, 1.0)
  rinv = lax.broadcast_in_dim(rinv1, (BLK_R, 1), (0,))
  mean = agg * lax.broadcast_in_dim(rinv1, (BLK_R, D), (0,))
  u = jnp.dot(emb_ref[...], Wr_ref[...], precision=_PREC)
  t = jnp.dot(mean, Wl_ref[...], precision=_PREC) + bl_ref[...] + u
  out_ref[...] = jnp.maximum(_l2norm(t), 0.0)
  rinv_ref[...] = rinv


def _tc_react(aggP, cnt8, emb_reaction, Wl, bl, Wr):
  grid = (R_PAD // BLK_R,)
  return pl.pallas_call(
      _tc_react_body,
      grid=grid,
      in_specs=[
          pl.BlockSpec((NC, BLK_R, D), lambda i: (0, i, 0)),
          pl.BlockSpec((CG, BLK_R), lambda i: (0, i)),
          pl.BlockSpec((1, D), lambda i: (0, 0)),
          pl.BlockSpec((D, D), lambda i: (0, 0)),
          pl.BlockSpec((D,), lambda i: (0,)),
          pl.BlockSpec((D, D), lambda i: (0, 0)),
      ],
      out_specs=[
          pl.BlockSpec((BLK_R, D), lambda i: (i, 0)),
          pl.BlockSpec((BLK_R, 1), lambda i: (i, 0)),
      ],
      out_shape=[
          jax.ShapeDtypeStruct((R_PAD, D), jnp.float32),
          jax.ShapeDtypeStruct((R_PAD, 1), jnp.float32),
      ],
  )(aggP, cnt8, emb_reaction, Wl, bl, Wr)


def _tc_final_body(bgg_ref, rinv_ref, hr1_ref, Wl_ref, bl_ref, Wr_ref,
                   Wo_ref, bo_ref, out_ref):
  agg = bgg_ref[0] + bgg_ref[1]
  mean = agg * rinv_ref[...]
  t = (jnp.dot(mean, Wl_ref[...], precision=_PREC) + bl_ref[...]
       + jnp.dot(hr1_ref[...], Wr_ref[...], precision=_PREC))
  h = jnp.maximum(_l2norm(t), 0.0)
  out_ref[...] = jnp.dot(h, Wo_ref[...], precision=_PREC) + bo_ref[...]


def _tc_final(bggP, rinv, h_r1, Wl, bl, Wr, W_out, b_out):
  grid = (R_PAD // BLK_R,)
  return pl.pallas_call(
      _tc_final_body,
      grid=grid,
      in_specs=[
          pl.BlockSpec((NC, BLK_R, D), lambda i: (0, i, 0)),
          pl.BlockSpec((BLK_R, 1), lambda i: (i, 0)),
          pl.BlockSpec((BLK_R, D), lambda i: (i, 0)),
          pl.BlockSpec((D, D), lambda i: (0, 0)),
          pl.BlockSpec((D,), lambda i: (0,)),
          pl.BlockSpec((D, D), lambda i: (0, 0)),
          pl.BlockSpec((D, OUT), lambda i: (0, 0)),
          pl.BlockSpec((OUT,), lambda i: (0,)),
      ],
      out_specs=pl.BlockSpec((BLK_R, OUT), lambda i: (i, 0)),
      out_shape=jax.ShapeDtypeStruct((R_PAD, OUT), jnp.float32),
  )(bggP, rinv, h_r1, Wl, bl, Wr, W_out, b_out)


def kernel(x_reaction, x_protein, edge_index_pr, edge_index_rp, emb_reaction,
           emb_protein, Wl_pr_0, bl_pr_0, Wr_pr_0, Wl_rp_0, bl_rp_0, Wr_rp_0,
           Wl_pr_1, bl_pr_1, Wr_pr_1, Wl_rp_1, bl_rp_1, Wr_rp_1, W_out, b_out):
  del x_reaction, Wl_rp_1, bl_rp_1, Wr_rp_1
  xp_pad = jnp.pad(x_protein[:, 0], (0, P_PAD - N_P))
  src_pr = edge_index_pr[0]
  dst_pr = edge_index_pr[1]
  dst_rp = edge_index_rp[1]
  npad = E_PAD - E
  pad_src = jnp.arange(npad, dtype=jnp.int32) % N_P
  pad_dst = N_R + (jnp.arange(npad, dtype=jnp.int32) % (R_PAD - N_R))
  src2d = jnp.concatenate([src_pr, pad_src]).reshape(ECH_P, K)
  dst2d = jnp.concatenate([dst_pr, pad_dst]).reshape(ECH_P, K)

  hp0 = _sc_hp0(xp_pad, emb_protein)
  aggP = _sc_edge_agg(hp0, src2d, dst2d).reshape(NC, R_PAD, D)
  flag, cnt8 = _sc_flags(dst_pr, dst_rp)
  flag = flag.reshape(FG, P_PAD)
  cnt8 = cnt8.reshape(CG, R_PAD)

  h_p1 = _tc_protein(hp0, flag, emb_reaction, Wl_rp_0, bl_rp_0, Wr_rp_0)
  h_r1, rinv = _tc_react(aggP, cnt8, emb_reaction, Wl_pr_0, bl_pr_0, Wr_pr_0)

  bggP = _sc_edge_agg(h_p1, src2d, dst2d).reshape(NC, R_PAD, D)
  out_pad = _tc_final(bggP, rinv, h_r1, Wl_pr_1, bl_pr_1, Wr_pr_1, W_out, b_out)
  return out_pad[:N_R]
